# Initial kernel scaffold; baseline (speedup 1.0000x reference)
import jax
import jax.numpy as jnp
from jax.experimental import pallas as pl


def kernel(query, value, mem_key, mem_value, q_w, q_b, v_w, v_b, out_w, out_b, ln1_g, ln1_b, ln3_g, ln3_b):
    raise NotImplementedError("write your pallas kernel here")


# speedup vs baseline: 2.0267x; 2.0267x over previous
import functools

import jax
import jax.numpy as jnp
from jax.experimental import pallas as pl
from jax.experimental.pallas import tpu as pltpu

N_SLOT = 112
N_HEAD = 8
DIM = 512
HEAD_DIM = 64
SLOT_PAD = 128
CAT = N_HEAD * SLOT_PAD
RADIUS = 16.0
EPS = 1e-5
TB = 256


def _f32dot(a, b):
    return jnp.dot(a, b, preferred_element_type=jnp.float32)


def _dot_rhs_t(a, b):
    return jax.lax.dot_general(a, b, (((1,), (1,)), ((), ())),
                               preferred_element_type=jnp.float32)


def _rownorm(x):
    ss = jnp.sum(x * x, axis=-1, keepdims=True)
    return x * jax.lax.rsqrt(jnp.maximum(ss, 1e-24))


def _ln(x, g, b):
    m = jnp.mean(x, axis=-1, keepdims=True)
    c = x - m
    v = jnp.mean(c * c, axis=-1, keepdims=True)
    return c * jax.lax.rsqrt(v + EPS) * g + b


def _prep_kernel(mk_ref, mvp_ref, owt_ref,
                 kpad_ref, wcat_ref, vnp_ref, closs_ref):
    kn = _rownorm(mk_ref[...])
    mvp = mvp_ref[...]
    lane = jax.lax.broadcasted_iota(jnp.int32, (SLOT_PAD, DIM), 1)
    for h in range(N_HEAD):
        kh = kn[h * N_SLOT:(h + 1) * N_SLOT, :]
        khp = jnp.concatenate(
            [kh, jnp.zeros((SLOT_PAD - N_SLOT, HEAD_DIM), jnp.float32)],
            axis=0)
        tiled = jnp.concatenate([khp] * N_HEAD, axis=1)
        kpad_ref[h * SLOT_PAD:(h + 1) * SLOT_PAD, :] = jnp.where(
            lane // HEAD_DIM == h, tiled, 0.0)
        wcat_ref[h * SLOT_PAD:(h + 1) * SLOT_PAD, :] = _f32dot(
            mvp, owt_ref[h * DIM:(h + 1) * DIM, :])
    vn = _rownorm(mvp)
    vnp_ref[...] = vn
    gram = _dot_rhs_t(vn, vn)
    r = jax.lax.broadcasted_iota(jnp.int32, (SLOT_PAD, SLOT_PAD), 0)
    c = jax.lax.broadcasted_iota(jnp.int32, (SLOT_PAD, SLOT_PAD), 1)
    eye = jnp.where((r == c) & (r < N_SLOT), 1.0, 0.0)
    closs_ref[...] = jnp.full((8, 128), jnp.sum(jnp.abs(eye - gram)) * 0.01,
                              jnp.float32)


def _main_kernel(q_ref, v_ref, qwt_ref, vwt_ref, kpad_ref, wcat_ref,
                 vnp_ref, mvp_ref, sel_ref, selt_ref, seg_ref, segt_ref,
                 bias_ref, fp_ref, ft_ref, part_ref):
    q = q_ref[...]
    v = v_ref[...]
    ln1_g, ln1_b = bias_ref[3:4, :], bias_ref[4:5, :]

    qp = _f32dot(q, qwt_ref[...]) + bias_ref[0:1, :]
    ssh = _f32dot(qp * qp, sel_ref[...])
    inv = jax.lax.rsqrt(jnp.maximum(ssh, 1e-24))
    qn = qp * _f32dot(inv, selt_ref[...])
    sim = _dot_rhs_t(qn, kpad_ref[...])
    lane = jax.lax.broadcasted_iota(jnp.int32, (TB, CAT), 1)
    e = jnp.where(lane % SLOT_PAD < N_SLOT, jnp.exp(RADIUS * sim), 0.0)
    ssum = _f32dot(e, seg_ref[...])
    rec = 1.0 / jnp.maximum(ssum, 1e-30)
    addr = e * _f32dot(rec, segt_ref[...])
    attn = _f32dot(addr, wcat_ref[...]) + bias_ref[2:3, :]
    fp_ref[...] = _ln(q + attn, ln1_g, ln1_b)

    vp = _f32dot(v, vwt_ref[...]) + bias_ref[1:2, :]
    sim2 = _dot_rhs_t(_rownorm(vp), vnp_ref[...])
    lane2 = jax.lax.broadcasted_iota(jnp.int32, (TB, SLOT_PAD), 1)
    e2 = jnp.where(lane2 < N_SLOT, jnp.exp(RADIUS * sim2), 0.0)
    addr2 = e2 / jnp.sum(e2, axis=-1, keepdims=True)
    ar = _f32dot(addr2, mvp_ref[...])
    cos = jnp.sum(_rownorm(ar) * _rownorm(v), axis=-1, keepdims=True)
    part_ref[...] = jnp.full((1, 1, 128), jnp.sum(jnp.abs(1.0 - cos)),
                             jnp.float32)
    ft_ref[...] = _ln(q + _ln(ar, bias_ref[5:6, :], bias_ref[6:7, :]),
                      ln1_g, ln1_b)


def kernel(query, value, mem_key, mem_value, q_w, q_b, v_w, v_b,
           out_w, out_b, ln1_g, ln1_b, ln3_g, ln3_b):
    B, S, C = query.shape
    N = B * S
    G = N // TB
    f32 = jnp.float32
    q2 = query.reshape(N, C)
    v2 = value.reshape(N, DIM)
    mvp = jnp.pad(mem_value, ((0, SLOT_PAD - N_SLOT), (0, 0)))
    owt = out_w.T

    kpad, wcat, vnp, closs_arr = pl.pallas_call(
        _prep_kernel,
        out_shape=[
            jax.ShapeDtypeStruct((CAT, DIM), f32),
            jax.ShapeDtypeStruct((CAT, DIM), f32),
            jax.ShapeDtypeStruct((SLOT_PAD, DIM), f32),
            jax.ShapeDtypeStruct((8, 128), f32),
        ],
        name="mem_prep",
    )(mem_key, mvp, owt)

    di = jnp.arange(DIM, dtype=jnp.int32)
    hi = jnp.arange(128, dtype=jnp.int32)
    ci = jnp.arange(CAT, dtype=jnp.int32)
    sel = (di[:, None] // HEAD_DIM == hi[None, :]).astype(f32)
    selt = sel.T
    seg = (ci[:, None] // SLOT_PAD == hi[None, :]).astype(f32)
    segt = seg.T
    bias_pack = jnp.stack(
        [q_b, v_b, out_b, ln1_g, ln1_b, ln3_g, ln3_b, jnp.zeros_like(q_b)],
        axis=0)

    res = lambda shape: pl.BlockSpec(shape, lambda i: (0,) * len(shape))
    fp, ft, parts = pl.pallas_call(
        _main_kernel,
        grid=(G,),
        in_specs=[
            pl.BlockSpec((TB, DIM), lambda i: (i, 0)),
            pl.BlockSpec((TB, DIM), lambda i: (i, 0)),
            res((DIM, DIM)),
            res((DIM, DIM)),
            res((CAT, DIM)),
            res((CAT, DIM)),
            res((SLOT_PAD, DIM)),
            res((SLOT_PAD, DIM)),
            res((DIM, 128)),
            res((128, DIM)),
            res((CAT, 128)),
            res((128, CAT)),
            res((8, DIM)),
        ],
        out_specs=[
            pl.BlockSpec((TB, DIM), lambda i: (i, 0)),
            pl.BlockSpec((TB, DIM), lambda i: (i, 0)),
            pl.BlockSpec((1, 1, 128), lambda i: (i, 0, 0)),
        ],
        out_shape=[
            jax.ShapeDtypeStruct((N, DIM), f32),
            jax.ShapeDtypeStruct((N, DIM), f32),
            jax.ShapeDtypeStruct((G, 1, 128), f32),
        ],
        compiler_params=pltpu.CompilerParams(
            dimension_semantics=("parallel",),
            vmem_limit_bytes=48 * 1024 * 1024,
        ),
        name="mem_main",
    )(q2, v2, q_w.T, v_w.T, kpad, wcat, vnp, mvp,
      sel, selt, seg, segt, bias_pack)

    f_predict = fp.reshape(B, S, C)
    f_target_recon = ft.reshape(B, S, C)
    recon_loss = jnp.sum(parts[:, 0, 0]) / N
    return (f_predict, f_target_recon, recon_loss, closs_arr[0, 0])


# speedup vs baseline: 2.4653x; 1.2164x over previous
import functools

import jax
import jax.numpy as jnp
from jax.experimental import pallas as pl
from jax.experimental.pallas import tpu as pltpu

N_SLOT = 112
N_HEAD = 8
DIM = 512
HEAD_DIM = 64
SLOT_PAD = 128
CAT = N_HEAD * SLOT_PAD
RADIUS = 16.0
EPS = 1e-5
TB = 512


def _f32dot(a, b):
    return jnp.dot(a, b, preferred_element_type=jnp.float32)


def _dot_rhs_t(a, b):
    return jax.lax.dot_general(a, b, (((1,), (1,)), ((), ())),
                               preferred_element_type=jnp.float32)


def _rownorm(x):
    ss = jnp.sum(x * x, axis=-1, keepdims=True)
    return x * jax.lax.rsqrt(jnp.maximum(ss, 1e-24))


def _ln(x, g, b):
    m = jnp.mean(x, axis=-1, keepdims=True)
    c = x - m
    v = jnp.mean(c * c, axis=-1, keepdims=True)
    return c * jax.lax.rsqrt(v + EPS) * g + b


def _prep_kernel(mk_ref, mvp_ref, owt_ref,
                 kpad_ref, wcat_ref, vnp_ref, closs_ref):
    kn = _rownorm(mk_ref[...])
    mvp = mvp_ref[...]
    lane = jax.lax.broadcasted_iota(jnp.int32, (SLOT_PAD, DIM), 1)
    for h in range(N_HEAD):
        kh = kn[h * N_SLOT:(h + 1) * N_SLOT, :]
        khp = jnp.concatenate(
            [kh, jnp.zeros((SLOT_PAD - N_SLOT, HEAD_DIM), jnp.float32)],
            axis=0)
        tiled = jnp.concatenate([khp] * N_HEAD, axis=1)
        kpad_ref[h * SLOT_PAD:(h + 1) * SLOT_PAD, :] = jnp.where(
            lane // HEAD_DIM == h, tiled, 0.0)
        wcat_ref[h * SLOT_PAD:(h + 1) * SLOT_PAD, :] = _f32dot(
            mvp, owt_ref[h * DIM:(h + 1) * DIM, :])
    vn = _rownorm(mvp)
    vnp_ref[...] = vn
    gram = _dot_rhs_t(vn, vn)
    r = jax.lax.broadcasted_iota(jnp.int32, (SLOT_PAD, SLOT_PAD), 0)
    c = jax.lax.broadcasted_iota(jnp.int32, (SLOT_PAD, SLOT_PAD), 1)
    eye = jnp.where((r == c) & (r < N_SLOT), 1.0, 0.0)
    closs_ref[...] = jnp.full((8, 128), jnp.sum(jnp.abs(eye - gram)) * 0.01,
                              jnp.float32)


def _main_kernel(q_ref, v_ref, qwt_ref, vwt_ref, kpad_ref, wcat_ref,
                 vnp_ref, mvp_ref, sel_ref, selt_ref, seg_ref, segt_ref,
                 bias_ref, fp_ref, ft_ref, part_ref):
    q = q_ref[...]
    v = v_ref[...]
    ln1_g, ln1_b = bias_ref[3:4, :], bias_ref[4:5, :]

    qp = _f32dot(q, qwt_ref[...]) + bias_ref[0:1, :]
    ssh = _f32dot(qp * qp, sel_ref[...])
    inv = jax.lax.rsqrt(jnp.maximum(ssh, 1e-24))
    qn = qp * _f32dot(inv, selt_ref[...])
    sim = _dot_rhs_t(qn, kpad_ref[...])
    lane = jax.lax.broadcasted_iota(jnp.int32, (TB, CAT), 1)
    e = jnp.where(lane % SLOT_PAD < N_SLOT, jnp.exp(RADIUS * sim), 0.0)
    ssum = _f32dot(e, seg_ref[...])
    rec = 1.0 / jnp.maximum(ssum, 1e-30)
    addr = e * _f32dot(rec, segt_ref[...])
    attn = _f32dot(addr, wcat_ref[...]) + bias_ref[2:3, :]
    fp_ref[...] = _ln(q + attn, ln1_g, ln1_b)

    vp = _f32dot(v, vwt_ref[...]) + bias_ref[1:2, :]
    sim2 = _dot_rhs_t(_rownorm(vp), vnp_ref[...])
    lane2 = jax.lax.broadcasted_iota(jnp.int32, (TB, SLOT_PAD), 1)
    e2 = jnp.where(lane2 < N_SLOT, jnp.exp(RADIUS * sim2), 0.0)
    addr2 = e2 / jnp.sum(e2, axis=-1, keepdims=True)
    ar = _f32dot(addr2, mvp_ref[...])
    cos = jnp.sum(_rownorm(ar) * _rownorm(v), axis=-1, keepdims=True)
    part_ref[...] = jnp.full((1, 1, 128), jnp.sum(jnp.abs(1.0 - cos)),
                             jnp.float32)
    ft_ref[...] = _ln(q + _ln(ar, bias_ref[5:6, :], bias_ref[6:7, :]),
                      ln1_g, ln1_b)


def kernel(query, value, mem_key, mem_value, q_w, q_b, v_w, v_b,
           out_w, out_b, ln1_g, ln1_b, ln3_g, ln3_b):
    B, S, C = query.shape
    N = B * S
    G = N // TB
    f32 = jnp.float32
    q2 = query.reshape(N, C)
    v2 = value.reshape(N, DIM)
    mvp = jnp.pad(mem_value, ((0, SLOT_PAD - N_SLOT), (0, 0)))
    owt = out_w.T

    kpad, wcat, vnp, closs_arr = pl.pallas_call(
        _prep_kernel,
        out_shape=[
            jax.ShapeDtypeStruct((CAT, DIM), f32),
            jax.ShapeDtypeStruct((CAT, DIM), f32),
            jax.ShapeDtypeStruct((SLOT_PAD, DIM), f32),
            jax.ShapeDtypeStruct((8, 128), f32),
        ],
        name="mem_prep",
    )(mem_key, mvp, owt)

    di = jnp.arange(DIM, dtype=jnp.int32)
    hi = jnp.arange(128, dtype=jnp.int32)
    ci = jnp.arange(CAT, dtype=jnp.int32)
    sel = (di[:, None] // HEAD_DIM == hi[None, :]).astype(f32)
    selt = sel.T
    seg = (ci[:, None] // SLOT_PAD == hi[None, :]).astype(f32)
    segt = seg.T
    bias_pack = jnp.stack(
        [q_b, v_b, out_b, ln1_g, ln1_b, ln3_g, ln3_b, jnp.zeros_like(q_b)],
        axis=0)

    res = lambda shape: pl.BlockSpec(shape, lambda i: (0,) * len(shape))
    fp, ft, parts = pl.pallas_call(
        _main_kernel,
        grid=(G,),
        in_specs=[
            pl.BlockSpec((TB, DIM), lambda i: (i, 0)),
            pl.BlockSpec((TB, DIM), lambda i: (i, 0)),
            res((DIM, DIM)),
            res((DIM, DIM)),
            res((CAT, DIM)),
            res((CAT, DIM)),
            res((SLOT_PAD, DIM)),
            res((SLOT_PAD, DIM)),
            res((DIM, 128)),
            res((128, DIM)),
            res((CAT, 128)),
            res((128, CAT)),
            res((8, DIM)),
        ],
        out_specs=[
            pl.BlockSpec((TB, DIM), lambda i: (i, 0)),
            pl.BlockSpec((TB, DIM), lambda i: (i, 0)),
            pl.BlockSpec((1, 1, 128), lambda i: (i, 0, 0)),
        ],
        out_shape=[
            jax.ShapeDtypeStruct((N, DIM), f32),
            jax.ShapeDtypeStruct((N, DIM), f32),
            jax.ShapeDtypeStruct((G, 1, 128), f32),
        ],
        compiler_params=pltpu.CompilerParams(
            dimension_semantics=("parallel",),
            vmem_limit_bytes=48 * 1024 * 1024,
        ),
        name="mem_main",
    )(q2, v2, q_w.T, v_w.T, kpad, wcat, vnp, mvp,
      sel, selt, seg, segt, bias_pack)

    f_predict = fp.reshape(B, S, C)
    f_target_recon = ft.reshape(B, S, C)
    recon_loss = jnp.sum(parts[:, 0, 0]) / N
    return (f_predict, f_target_recon, recon_loss, closs_arr[0, 0])


# speedup vs baseline: 2.5500x; 1.0344x over previous
import functools

import numpy as np

import jax
import jax.numpy as jnp
from jax.experimental import pallas as pl
from jax.experimental.pallas import tpu as pltpu

N_SLOT = 112
N_HEAD = 8
DIM = 512
HEAD_DIM = 64
SLOT_PAD = 128
CAT = N_HEAD * SLOT_PAD
RADIUS = 16.0
EPS = 1e-5
TB = 512


def _f32dot(a, b):
    return jnp.dot(a, b, preferred_element_type=jnp.float32)


def _dot_rhs_t(a, b):
    return jax.lax.dot_general(a, b, (((1,), (1,)), ((), ())),
                               preferred_element_type=jnp.float32)


def _rownorm(x):
    ss = jnp.sum(x * x, axis=-1, keepdims=True)
    return x * jax.lax.rsqrt(jnp.maximum(ss, 1e-24))


def _ln(x, g, b):
    m = jnp.mean(x, axis=-1, keepdims=True)
    c = x - m
    v = jnp.mean(c * c, axis=-1, keepdims=True)
    return c * jax.lax.rsqrt(v + EPS) * g + b


def _prep_kernel(mk_ref, mvp_ref, ow_ref,
                 kpad_ref, wcat_ref, vnp_ref, closs_ref):
    kn = _rownorm(mk_ref[...])
    mvp = mvp_ref[...]
    lane = jax.lax.broadcasted_iota(jnp.int32, (SLOT_PAD, DIM), 1)
    for h in range(N_HEAD):
        kh = kn[h * N_SLOT:(h + 1) * N_SLOT, :]
        khp = jnp.concatenate(
            [kh, jnp.zeros((SLOT_PAD - N_SLOT, HEAD_DIM), jnp.float32)],
            axis=0)
        tiled = jnp.concatenate([khp] * N_HEAD, axis=1)
        kpad_ref[h * SLOT_PAD:(h + 1) * SLOT_PAD, :] = jnp.where(
            lane // HEAD_DIM == h, tiled, 0.0)
        wcat_ref[h * SLOT_PAD:(h + 1) * SLOT_PAD, :] = _dot_rhs_t(
            mvp, ow_ref[:, h * DIM:(h + 1) * DIM]).astype(jnp.bfloat16)
    vn = _rownorm(mvp)
    vnp_ref[...] = vn
    gram = _dot_rhs_t(vn, vn)
    r = jax.lax.broadcasted_iota(jnp.int32, (SLOT_PAD, SLOT_PAD), 0)
    c = jax.lax.broadcasted_iota(jnp.int32, (SLOT_PAD, SLOT_PAD), 1)
    eye = jnp.where((r == c) & (r < N_SLOT), 1.0, 0.0)
    closs_ref[...] = jnp.full((8, 128), jnp.sum(jnp.abs(eye - gram)) * 0.01,
                              jnp.float32)


def _main_kernel(q_ref, v_ref, qw_ref, vw_ref, kpad_ref, wcat_ref,
                 vnp_ref, mvp_ref, sel_ref, selt_ref, seg_ref, segt_ref,
                 bias_ref, fp_ref, ft_ref, part_ref):
    bf16 = jnp.bfloat16
    q = q_ref[...]
    v = v_ref[...]
    ln1_g, ln1_b = bias_ref[3:4, :], bias_ref[4:5, :]

    qp = _dot_rhs_t(q.astype(bf16), qw_ref[...]) + bias_ref[0:1, :]
    qps = qp * qp
    ssh = _f32dot(qps.astype(bf16), sel_ref[...])
    inv = jax.lax.rsqrt(jnp.maximum(ssh, 1e-24))
    qn = qp * _f32dot(inv, selt_ref[...])
    sim = _dot_rhs_t(qn, kpad_ref[...])
    lane = jax.lax.broadcasted_iota(jnp.int32, (TB, CAT), 1)
    e = jnp.where(lane % SLOT_PAD < N_SLOT, jnp.exp(RADIUS * sim), 0.0)
    ssum = _f32dot(e.astype(bf16), seg_ref[...])
    rec = 1.0 / jnp.maximum(ssum, 1e-30)
    addr = e * _f32dot(rec, segt_ref[...])
    attn = _f32dot(addr.astype(bf16), wcat_ref[...]) + bias_ref[2:3, :]
    fp_ref[...] = _ln(q + attn, ln1_g, ln1_b)

    vp = _dot_rhs_t(v.astype(bf16), vw_ref[...]) + bias_ref[1:2, :]
    sim2 = _dot_rhs_t(_rownorm(vp), vnp_ref[...])
    lane2 = jax.lax.broadcasted_iota(jnp.int32, (TB, SLOT_PAD), 1)
    e2 = jnp.where(lane2 < N_SLOT, jnp.exp(RADIUS * sim2), 0.0)
    addr2 = e2 / jnp.sum(e2, axis=-1, keepdims=True)
    ar = _f32dot(addr2, mvp_ref[...])
    cos = jnp.sum(_rownorm(ar) * _rownorm(v), axis=-1, keepdims=True)
    part_ref[...] = jnp.full((1, 1, 128), jnp.sum(jnp.abs(1.0 - cos)),
                             jnp.float32)
    ft_ref[...] = _ln(q + _ln(ar, bias_ref[5:6, :], bias_ref[6:7, :]),
                      ln1_g, ln1_b)


def kernel(query, value, mem_key, mem_value, q_w, q_b, v_w, v_b,
           out_w, out_b, ln1_g, ln1_b, ln3_g, ln3_b):
    B, S, C = query.shape
    N = B * S
    G = N // TB
    f32 = jnp.float32
    q2 = query.reshape(N, C)
    v2 = value.reshape(N, DIM)
    mvp = jnp.pad(mem_value, ((0, SLOT_PAD - N_SLOT), (0, 0)))

    kpad, wcat, vnp, closs_arr = pl.pallas_call(
        _prep_kernel,
        out_shape=[
            jax.ShapeDtypeStruct((CAT, DIM), f32),
            jax.ShapeDtypeStruct((CAT, DIM), jnp.bfloat16),
            jax.ShapeDtypeStruct((SLOT_PAD, DIM), f32),
            jax.ShapeDtypeStruct((8, 128), f32),
        ],
        name="mem_prep",
    )(mem_key, mvp, out_w)

    di = np.arange(DIM)
    hi = np.arange(128)
    ci = np.arange(CAT)
    sel = jnp.asarray((di[:, None] // HEAD_DIM == hi[None, :]),
                      jnp.bfloat16)
    selt = jnp.asarray((hi[:, None] == di[None, :] // HEAD_DIM), f32)
    seg = jnp.asarray((ci[:, None] // SLOT_PAD == hi[None, :]),
                      jnp.bfloat16)
    segt = jnp.asarray((hi[:, None] == ci[None, :] // SLOT_PAD), f32)
    bias_pack = jnp.stack(
        [q_b, v_b, out_b, ln1_g, ln1_b, ln3_g, ln3_b, jnp.zeros_like(q_b)],
        axis=0)

    res = lambda shape: pl.BlockSpec(shape, lambda i: (0,) * len(shape))
    fp, ft, parts = pl.pallas_call(
        _main_kernel,
        grid=(G,),
        in_specs=[
            pl.BlockSpec((TB, DIM), lambda i: (i, 0)),
            pl.BlockSpec((TB, DIM), lambda i: (i, 0)),
            res((DIM, DIM)),
            res((DIM, DIM)),
            res((CAT, DIM)),
            res((CAT, DIM)),
            res((SLOT_PAD, DIM)),
            res((SLOT_PAD, DIM)),
            res((DIM, 128)),
            res((128, DIM)),
            res((CAT, 128)),
            res((128, CAT)),
            res((8, DIM)),
        ],
        out_specs=[
            pl.BlockSpec((TB, DIM), lambda i: (i, 0)),
            pl.BlockSpec((TB, DIM), lambda i: (i, 0)),
            pl.BlockSpec((1, 1, 128), lambda i: (i, 0, 0)),
        ],
        out_shape=[
            jax.ShapeDtypeStruct((N, DIM), f32),
            jax.ShapeDtypeStruct((N, DIM), f32),
            jax.ShapeDtypeStruct((G, 1, 128), f32),
        ],
        compiler_params=pltpu.CompilerParams(
            dimension_semantics=("parallel",),
            vmem_limit_bytes=48 * 1024 * 1024,
        ),
        name="mem_main",
    )(q2, v2, q_w.astype(jnp.bfloat16), v_w.astype(jnp.bfloat16),
      kpad, wcat, vnp, mvp, sel, selt, seg, segt, bias_pack)

    f_predict = fp.reshape(B, S, C)
    f_target_recon = ft.reshape(B, S, C)
    recon_loss = jnp.sum(parts[:, 0, 0]) / N
    return (f_predict, f_target_recon, recon_loss, closs_arr[0, 0])


# speedup vs baseline: 2.8096x; 1.1018x over previous
import functools

import numpy as np

import jax
import jax.numpy as jnp
from jax.experimental import pallas as pl
from jax.experimental.pallas import tpu as pltpu

N_SLOT = 112
N_HEAD = 8
DIM = 512
HEAD_DIM = 64
SLOT_PAD = 128
CAT = N_HEAD * SLOT_PAD
RADIUS = 16.0
EPS = 1e-5
TB = 512


def _f32dot(a, b):
    return jnp.dot(a, b, preferred_element_type=jnp.float32)


def _dot_rhs_t(a, b):
    return jax.lax.dot_general(a, b, (((1,), (1,)), ((), ())),
                               preferred_element_type=jnp.float32)


def _rownorm(x):
    ss = jnp.sum(x * x, axis=-1, keepdims=True)
    return x * jax.lax.rsqrt(jnp.maximum(ss, 1e-24))


def _ln(x, g, b):
    m = jnp.mean(x, axis=-1, keepdims=True)
    m2 = jnp.mean(x * x, axis=-1, keepdims=True)
    r = jax.lax.rsqrt(jnp.maximum(m2 - m * m, 0.0) + EPS)
    return (x - m) * r * g + b


def _prep_kernel(mk_ref, mvp_ref, ow_ref,
                 kpad_ref, wcat_ref, vnp_ref, closs_ref):
    kn = _rownorm(mk_ref[...])
    mvp = mvp_ref[...]
    lane = jax.lax.broadcasted_iota(jnp.int32, (SLOT_PAD, DIM), 1)
    for h in range(N_HEAD):
        kh = kn[h * N_SLOT:(h + 1) * N_SLOT, :]
        khp = jnp.concatenate(
            [kh, jnp.zeros((SLOT_PAD - N_SLOT, HEAD_DIM), jnp.float32)],
            axis=0)
        tiled = jnp.concatenate([khp] * N_HEAD, axis=1)
        kpad_ref[h * SLOT_PAD:(h + 1) * SLOT_PAD, :] = jnp.where(
            lane // HEAD_DIM == h, tiled, 0.0).astype(jnp.bfloat16)
        wcat_ref[h * SLOT_PAD:(h + 1) * SLOT_PAD, :] = _dot_rhs_t(
            mvp, ow_ref[:, h * DIM:(h + 1) * DIM]).astype(jnp.bfloat16)
    vn = _rownorm(mvp)
    vnp_ref[...] = vn.astype(jnp.bfloat16)
    gram = _dot_rhs_t(vn, vn)
    r = jax.lax.broadcasted_iota(jnp.int32, (SLOT_PAD, SLOT_PAD), 0)
    c = jax.lax.broadcasted_iota(jnp.int32, (SLOT_PAD, SLOT_PAD), 1)
    eye = jnp.where((r == c) & (r < N_SLOT), 1.0, 0.0)
    closs_ref[...] = jnp.full((8, 128), jnp.sum(jnp.abs(eye - gram)) * 0.01,
                              jnp.float32)


def _main_kernel(q_ref, v_ref, qw_ref, vw_ref, kpad_ref, wcat_ref,
                 vnp_ref, mvp_ref, sel_ref, selt_ref, seg_ref, segt_ref,
                 bias_ref, fp_ref, ft_ref, part_ref):
    bf16 = jnp.bfloat16
    q = q_ref[...]
    v = v_ref[...]
    ln1_g, ln1_b = bias_ref[3:4, :], bias_ref[4:5, :]

    qp = _dot_rhs_t(q.astype(bf16), qw_ref[...]) + bias_ref[0:1, :]
    qps = qp * qp
    ssh = _f32dot(qps.astype(bf16), sel_ref[...])
    inv = jax.lax.rsqrt(jnp.maximum(ssh, 1e-24))
    qn = qp * _f32dot(inv, selt_ref[...])
    sim = _dot_rhs_t(qn.astype(bf16), kpad_ref[...])
    e = jnp.exp(RADIUS * sim)
    ssum = _f32dot(e.astype(bf16), seg_ref[...]) - 16.0
    rec = 1.0 / jnp.maximum(ssum, 1e-30)
    addr = e * _f32dot(rec, segt_ref[...])
    attn = _f32dot(addr.astype(bf16), wcat_ref[...]) + bias_ref[2:3, :]
    fp_ref[...] = _ln(q + attn, ln1_g, ln1_b)

    vp = _dot_rhs_t(v.astype(bf16), vw_ref[...]) + bias_ref[1:2, :]
    sim2 = _dot_rhs_t(_rownorm(vp).astype(bf16), vnp_ref[...])
    e2 = jnp.exp(RADIUS * sim2)
    addr2 = e2 / (jnp.sum(e2, axis=-1, keepdims=True) - 16.0)
    ar = _f32dot(addr2, mvp_ref[...])
    cos = jnp.sum(_rownorm(ar) * _rownorm(v), axis=-1, keepdims=True)
    part_ref[...] = jnp.full((1, 1, 128), jnp.sum(jnp.abs(1.0 - cos)),
                             jnp.float32)
    ft_ref[...] = _ln(q + _ln(ar, bias_ref[5:6, :], bias_ref[6:7, :]),
                      ln1_g, ln1_b)


def kernel(query, value, mem_key, mem_value, q_w, q_b, v_w, v_b,
           out_w, out_b, ln1_g, ln1_b, ln3_g, ln3_b):
    B, S, C = query.shape
    N = B * S
    G = N // TB
    f32 = jnp.float32
    q2 = query.reshape(N, C)
    v2 = value.reshape(N, DIM)
    mvp = jnp.pad(mem_value, ((0, SLOT_PAD - N_SLOT), (0, 0)))

    kpad, wcat, vnp, closs_arr = pl.pallas_call(
        _prep_kernel,
        out_shape=[
            jax.ShapeDtypeStruct((CAT, DIM), jnp.bfloat16),
            jax.ShapeDtypeStruct((CAT, DIM), jnp.bfloat16),
            jax.ShapeDtypeStruct((SLOT_PAD, DIM), jnp.bfloat16),
            jax.ShapeDtypeStruct((8, 128), f32),
        ],
        name="mem_prep",
    )(mem_key, mvp, out_w)

    di = np.arange(DIM)
    hi = np.arange(128)
    ci = np.arange(CAT)
    sel = jnp.asarray((di[:, None] // HEAD_DIM == hi[None, :]),
                      jnp.bfloat16)
    selt = jnp.asarray((hi[:, None] == di[None, :] // HEAD_DIM), f32)
    seg = jnp.asarray((ci[:, None] // SLOT_PAD == hi[None, :]),
                      jnp.bfloat16)
    segt = jnp.asarray((hi[:, None] == ci[None, :] // SLOT_PAD), f32)
    bias_pack = jnp.stack(
        [q_b, v_b, out_b, ln1_g, ln1_b, ln3_g, ln3_b, jnp.zeros_like(q_b)],
        axis=0)

    res = lambda shape: pl.BlockSpec(shape, lambda i: (0,) * len(shape))
    blk = lambda i: (i, 0)
    fp, ft, parts = pl.pallas_call(
        _main_kernel,
        grid=(G,),
        in_specs=[
            pl.BlockSpec((TB, DIM), blk),
            pl.BlockSpec((TB, DIM), blk),
            res((DIM, DIM)),
            res((DIM, DIM)),
            res((CAT, DIM)),
            res((CAT, DIM)),
            res((SLOT_PAD, DIM)),
            res((SLOT_PAD, DIM)),
            res((DIM, 128)),
            res((128, DIM)),
            res((CAT, 128)),
            res((128, CAT)),
            res((8, DIM)),
        ],
        out_specs=[
            pl.BlockSpec((TB, DIM), blk),
            pl.BlockSpec((TB, DIM), blk),
            pl.BlockSpec((1, 1, 128), lambda i: (i, 0, 0)),
        ],
        out_shape=[
            jax.ShapeDtypeStruct((N, DIM), f32),
            jax.ShapeDtypeStruct((N, DIM), f32),
            jax.ShapeDtypeStruct((G, 1, 128), f32),
        ],
        compiler_params=pltpu.CompilerParams(
            dimension_semantics=("parallel",),
            vmem_limit_bytes=48 * 1024 * 1024,
        ),
        name="mem_main",
    )(q2, v2, q_w.astype(jnp.bfloat16), v_w.astype(jnp.bfloat16),
      kpad, wcat, vnp, mvp, sel, selt, seg, segt, bias_pack)

    f_predict = fp.reshape(B, S, C)
    f_target_recon = ft.reshape(B, S, C)
    recon_loss = jnp.sum(parts[:, 0, 0]) / N
    return (f_predict, f_target_recon, recon_loss, closs_arr[0, 0])


# speedup vs baseline: 2.8976x; 1.0313x over previous
import functools

import numpy as np

import jax
import jax.numpy as jnp
from jax.experimental import pallas as pl
from jax.experimental.pallas import tpu as pltpu

N_SLOT = 112
N_HEAD = 8
DIM = 512
HEAD_DIM = 64
SLOT_PAD = 128
CAT = N_HEAD * SLOT_PAD
RADIUS = 16.0
EPS = 1e-5
TB = 512


def _f32dot(a, b):
    return jnp.dot(a, b, preferred_element_type=jnp.float32)


def _dot_rhs_t(a, b):
    return jax.lax.dot_general(a, b, (((1,), (1,)), ((), ())),
                               preferred_element_type=jnp.float32)


def _rownorm(x):
    ss = jnp.sum(x * x, axis=-1, keepdims=True)
    return x * jax.lax.rsqrt(jnp.maximum(ss, 1e-24))


def _ln(x, g, b):
    m = jnp.mean(x, axis=-1, keepdims=True)
    m2 = jnp.mean(x * x, axis=-1, keepdims=True)
    r = jax.lax.rsqrt(jnp.maximum(m2 - m * m, 0.0) + EPS)
    return (x - m) * r * g + b


def _prep_kernel(mk_ref, mvp_ref, ow_ref,
                 kpad_ref, wcat_ref, vnp_ref, closs_ref):
    kn = _rownorm(mk_ref[...])
    mvp = mvp_ref[...]
    lane = jax.lax.broadcasted_iota(jnp.int32, (SLOT_PAD, DIM), 1)
    for h in range(N_HEAD):
        kh = kn[h * N_SLOT:(h + 1) * N_SLOT, :]
        khp = jnp.concatenate(
            [kh, jnp.zeros((SLOT_PAD - N_SLOT, HEAD_DIM), jnp.float32)],
            axis=0)
        tiled = jnp.concatenate([khp] * N_HEAD, axis=1)
        kpad_ref[h * SLOT_PAD:(h + 1) * SLOT_PAD, :] = jnp.where(
            lane // HEAD_DIM == h, tiled, 0.0).astype(jnp.bfloat16)
        wcat_ref[h * SLOT_PAD:(h + 1) * SLOT_PAD, :] = _dot_rhs_t(
            mvp, ow_ref[:, h * DIM:(h + 1) * DIM]).astype(jnp.bfloat16)
    vn = _rownorm(mvp)
    vnp_ref[...] = vn.astype(jnp.bfloat16)
    gram = _dot_rhs_t(vn, vn)
    r = jax.lax.broadcasted_iota(jnp.int32, (SLOT_PAD, SLOT_PAD), 0)
    c = jax.lax.broadcasted_iota(jnp.int32, (SLOT_PAD, SLOT_PAD), 1)
    eye = jnp.where((r == c) & (r < N_SLOT), 1.0, 0.0)
    closs_ref[...] = jnp.full((8, 128), jnp.sum(jnp.abs(eye - gram)) * 0.01,
                              jnp.float32)


def _main_kernel(q_ref, v_ref, qw_ref, vw_ref, kpad_ref, wcat_ref,
                 vnp_ref, mvp_ref, sel_ref, segt_bf_ref, seg_ref, segt_ref,
                 bias_ref, fp_ref, ft_ref, part_ref):
    bf16 = jnp.bfloat16
    LOG2E = 1.4426950408889634
    q = q_ref[...]
    v = v_ref[...]
    ln1_g, ln1_b = bias_ref[3:4, :], bias_ref[4:5, :]

    qp = _dot_rhs_t(q.astype(bf16), qw_ref[...]) + bias_ref[0:1, :]
    qpb = qp.astype(bf16)
    ssh = _f32dot(qpb * qpb, sel_ref[...])
    invc = jax.lax.rsqrt(jnp.maximum(ssh, 1e-24)) * (RADIUS * LOG2E)
    raw = _dot_rhs_t(qpb, kpad_ref[...])
    e = jnp.exp2(raw * _f32dot(invc, segt_ref[...]))
    eb = e.astype(bf16)
    ssum = _f32dot(eb, seg_ref[...]) - 16.0
    rec = (1.0 / jnp.maximum(ssum, 1e-30)).astype(bf16)
    addr = eb * _f32dot(rec, segt_bf_ref[...]).astype(bf16)
    attn = _f32dot(addr, wcat_ref[...]) + bias_ref[2:3, :]
    fp_ref[...] = _ln(q + attn, ln1_g, ln1_b)

    vp = _dot_rhs_t(v.astype(bf16), vw_ref[...]) + bias_ref[1:2, :]
    ssv = jnp.sum(vp * vp, axis=-1, keepdims=True)
    rc = jax.lax.rsqrt(jnp.maximum(ssv, 1e-24)) * (RADIUS * LOG2E)
    raw2 = _dot_rhs_t(vp.astype(bf16), vnp_ref[...])
    e2 = jnp.exp2(raw2 * rc)
    addr2 = e2 / (jnp.sum(e2, axis=-1, keepdims=True) - 16.0)
    ar = _f32dot(addr2.astype(bf16), mvp_ref[...])
    s_av = jnp.sum(ar * v, axis=-1, keepdims=True)
    s_aa = jnp.sum(ar * ar, axis=-1, keepdims=True)
    s_vv = jnp.sum(v * v, axis=-1, keepdims=True)
    cos = s_av * jax.lax.rsqrt(jnp.maximum(s_aa * s_vv, 1e-24))
    part_ref[...] = jnp.full((1, 1, 128), jnp.sum(jnp.abs(1.0 - cos)),
                             jnp.float32)
    ft_ref[...] = _ln(q + _ln(ar, bias_ref[5:6, :], bias_ref[6:7, :]),
                      ln1_g, ln1_b)


def kernel(query, value, mem_key, mem_value, q_w, q_b, v_w, v_b,
           out_w, out_b, ln1_g, ln1_b, ln3_g, ln3_b):
    B, S, C = query.shape
    N = B * S
    G = N // TB
    f32 = jnp.float32
    q2 = query.reshape(N, C)
    v2 = value.reshape(N, DIM)
    mvp = jnp.pad(mem_value, ((0, SLOT_PAD - N_SLOT), (0, 0)))

    kpad, wcat, vnp, closs_arr = pl.pallas_call(
        _prep_kernel,
        out_shape=[
            jax.ShapeDtypeStruct((CAT, DIM), jnp.bfloat16),
            jax.ShapeDtypeStruct((CAT, DIM), jnp.bfloat16),
            jax.ShapeDtypeStruct((SLOT_PAD, DIM), jnp.bfloat16),
            jax.ShapeDtypeStruct((8, 128), f32),
        ],
        name="mem_prep",
    )(mem_key, mvp, out_w)

    di = np.arange(DIM)
    hi = np.arange(128)
    ci = np.arange(CAT)
    sel = jnp.asarray((di[:, None] // HEAD_DIM == hi[None, :]),
                      jnp.bfloat16)
    seg = jnp.asarray((ci[:, None] // SLOT_PAD == hi[None, :]),
                      jnp.bfloat16)
    segt_np = hi[:, None] == ci[None, :] // SLOT_PAD
    segt = jnp.asarray(segt_np, f32)
    segt_bf = jnp.asarray(segt_np, jnp.bfloat16)
    bias_pack = jnp.stack(
        [q_b, v_b, out_b, ln1_g, ln1_b, ln3_g, ln3_b, jnp.zeros_like(q_b)],
        axis=0)

    res = lambda shape: pl.BlockSpec(shape, lambda i: (0,) * len(shape))
    blk = lambda i: (i, 0)
    fp, ft, parts = pl.pallas_call(
        _main_kernel,
        grid=(G,),
        in_specs=[
            pl.BlockSpec((TB, DIM), blk),
            pl.BlockSpec((TB, DIM), blk),
            res((DIM, DIM)),
            res((DIM, DIM)),
            res((CAT, DIM)),
            res((CAT, DIM)),
            res((SLOT_PAD, DIM)),
            res((SLOT_PAD, DIM)),
            res((DIM, 128)),
            res((128, CAT)),
            res((CAT, 128)),
            res((128, CAT)),
            res((8, DIM)),
        ],
        out_specs=[
            pl.BlockSpec((TB, DIM), blk),
            pl.BlockSpec((TB, DIM), blk),
            pl.BlockSpec((1, 1, 128), lambda i: (i, 0, 0)),
        ],
        out_shape=[
            jax.ShapeDtypeStruct((N, DIM), f32),
            jax.ShapeDtypeStruct((N, DIM), f32),
            jax.ShapeDtypeStruct((G, 1, 128), f32),
        ],
        compiler_params=pltpu.CompilerParams(
            dimension_semantics=("parallel",),
            vmem_limit_bytes=48 * 1024 * 1024,
        ),
        name="mem_main",
    )(q2, v2, q_w.astype(jnp.bfloat16), v_w.astype(jnp.bfloat16),
      kpad, wcat, vnp, mvp.astype(jnp.bfloat16), sel, segt_bf, seg, segt,
      bias_pack)

    f_predict = fp.reshape(B, S, C)
    f_target_recon = ft.reshape(B, S, C)
    recon_loss = jnp.sum(parts[:, 0, 0]) / N
    return (f_predict, f_target_recon, recon_loss, closs_arr[0, 0])


# speedup vs baseline: 2.9737x; 1.0263x over previous
import functools

import numpy as np

import jax
import jax.numpy as jnp
from jax.experimental import pallas as pl
from jax.experimental.pallas import tpu as pltpu

N_SLOT = 112
N_HEAD = 8
DIM = 512
HEAD_DIM = 64
SLOT_PAD = 128
CAT = N_HEAD * SLOT_PAD
RADIUS = 16.0
EPS = 1e-5
TB = 512


def _f32dot(a, b):
    return jnp.dot(a, b, preferred_element_type=jnp.float32)


def _dot_rhs_t(a, b):
    return jax.lax.dot_general(a, b, (((1,), (1,)), ((), ())),
                               preferred_element_type=jnp.float32)


def _rownorm(x):
    ss = jnp.sum(x * x, axis=-1, keepdims=True)
    return x * jax.lax.rsqrt(jnp.maximum(ss, 1e-24))


def _ln(x, g, b):
    m = jnp.mean(x, axis=-1, keepdims=True)
    m2 = jnp.mean(x * x, axis=-1, keepdims=True)
    r = jax.lax.rsqrt(jnp.maximum(m2 - m * m, 0.0) + EPS)
    return (x - m) * r * g + b


def _prep_kernel(mkt_ref, mvp_ref, mvpt_ref, ow_ref,
                 kpadt_ref, wcat_ref, vnpt_ref, closs_ref):
    mkt = mkt_ref[...]
    sst = jnp.sum(mkt * mkt, axis=0, keepdims=True)
    knt = mkt * jax.lax.rsqrt(jnp.maximum(sst, 1e-24))
    mvp = mvp_ref[...]
    for h in range(N_HEAD):
        bh = knt[:, h * N_SLOT:(h + 1) * N_SLOT]
        kpadt_ref[h * HEAD_DIM:(h + 1) * HEAD_DIM, :] = jnp.pad(
            bh, ((0, 0), (h * SLOT_PAD, CAT - h * SLOT_PAD - N_SLOT)),
        ).astype(jnp.bfloat16)
        wcat_ref[h * SLOT_PAD:(h + 1) * SLOT_PAD, :] = _dot_rhs_t(
            mvp, ow_ref[:, h * DIM:(h + 1) * DIM]).astype(jnp.bfloat16)
    mvpt = mvpt_ref[...]
    ssv = jnp.sum(mvpt * mvpt, axis=0, keepdims=True)
    vnt = mvpt * jax.lax.rsqrt(jnp.maximum(ssv, 1e-24))
    vnpt_ref[...] = vnt.astype(jnp.bfloat16)
    gram = jax.lax.dot_general(vnt, vnt, (((0,), (0,)), ((), ())),
                               preferred_element_type=jnp.float32)
    r = jax.lax.broadcasted_iota(jnp.int32, (SLOT_PAD, SLOT_PAD), 0)
    c = jax.lax.broadcasted_iota(jnp.int32, (SLOT_PAD, SLOT_PAD), 1)
    eye = jnp.where((r == c) & (r < N_SLOT), 1.0, 0.0)
    closs_ref[...] = jnp.full((8, 128), jnp.sum(jnp.abs(eye - gram)) * 0.01,
                              jnp.float32)


def _main_kernel(q_ref, v_ref, qwt_ref, vwt_ref, kpadt_ref, wcat_ref,
                 vnpt_ref, mvp_ref, sel_ref, segt_bf_ref, seg_ref, segt_ref,
                 bias_ref, fp_ref, ft_ref, part_ref):
    bf16 = jnp.bfloat16
    LOG2E = 1.4426950408889634
    q = q_ref[...]
    v = v_ref[...]
    ln1_g, ln1_b = bias_ref[3:4, :], bias_ref[4:5, :]

    qp = _f32dot(q.astype(bf16), qwt_ref[...]) + bias_ref[0:1, :]
    qpb = qp.astype(bf16)
    ssh = _f32dot(qpb * qpb, sel_ref[...])
    invc = jax.lax.rsqrt(jnp.maximum(ssh, 1e-24)) * (RADIUS * LOG2E)
    raw = _f32dot(qpb, kpadt_ref[...])
    e = jnp.exp2(raw * _f32dot(invc, segt_ref[...]))
    eb = e.astype(bf16)
    ssum = _f32dot(eb, seg_ref[...]) - 16.0
    rec = (1.0 / jnp.maximum(ssum, 1e-30)).astype(bf16)
    addr = eb * _f32dot(rec, segt_bf_ref[...]).astype(bf16)
    attn = _f32dot(addr, wcat_ref[...]) + bias_ref[2:3, :]
    fp_ref[...] = _ln(q + attn, ln1_g, ln1_b)

    vp = _f32dot(v.astype(bf16), vwt_ref[...]) + bias_ref[1:2, :]
    ssv = jnp.sum(vp * vp, axis=-1, keepdims=True)
    rc = jax.lax.rsqrt(jnp.maximum(ssv, 1e-24)) * (RADIUS * LOG2E)
    raw2 = _f32dot(vp.astype(bf16), vnpt_ref[...])
    e2 = jnp.exp2(raw2 * rc)
    addr2 = e2 / (jnp.sum(e2, axis=-1, keepdims=True) - 16.0)
    ar = _f32dot(addr2.astype(bf16), mvp_ref[...])
    s_av = jnp.sum(ar * v, axis=-1, keepdims=True)
    s_aa = jnp.sum(ar * ar, axis=-1, keepdims=True)
    s_vv = jnp.sum(v * v, axis=-1, keepdims=True)
    cos = s_av * jax.lax.rsqrt(jnp.maximum(s_aa * s_vv, 1e-24))
    part_ref[...] = jnp.full((1, 1, 128), jnp.sum(jnp.abs(1.0 - cos)),
                             jnp.float32)
    ft_ref[...] = _ln(q + _ln(ar, bias_ref[5:6, :], bias_ref[6:7, :]),
                      ln1_g, ln1_b)


def kernel(query, value, mem_key, mem_value, q_w, q_b, v_w, v_b,
           out_w, out_b, ln1_g, ln1_b, ln3_g, ln3_b):
    B, S, C = query.shape
    N = B * S
    G = N // TB
    f32 = jnp.float32
    q2 = query.reshape(N, C)
    v2 = value.reshape(N, DIM)
    mvp = jnp.pad(mem_value, ((0, SLOT_PAD - N_SLOT), (0, 0)))

    kpadt, wcat, vnpt, closs_arr = pl.pallas_call(
        _prep_kernel,
        out_shape=[
            jax.ShapeDtypeStruct((DIM, CAT), jnp.bfloat16),
            jax.ShapeDtypeStruct((CAT, DIM), jnp.bfloat16),
            jax.ShapeDtypeStruct((DIM, SLOT_PAD), jnp.bfloat16),
            jax.ShapeDtypeStruct((8, 128), f32),
        ],
        name="mem_prep",
    )(mem_key.T, mvp, mvp.T, out_w)

    di = np.arange(DIM)
    hi = np.arange(128)
    ci = np.arange(CAT)
    sel = jnp.asarray((di[:, None] // HEAD_DIM == hi[None, :]),
                      jnp.bfloat16)
    seg = jnp.asarray((ci[:, None] // SLOT_PAD == hi[None, :]),
                      jnp.bfloat16)
    segt_np = hi[:, None] == ci[None, :] // SLOT_PAD
    segt = jnp.asarray(segt_np, f32)
    segt_bf = jnp.asarray(segt_np, jnp.bfloat16)
    bias_pack = jnp.stack(
        [q_b, v_b, out_b, ln1_g, ln1_b, ln3_g, ln3_b, jnp.zeros_like(q_b)],
        axis=0)

    res = lambda shape: pl.BlockSpec(shape, lambda i: (0,) * len(shape))
    blk = lambda i: (i, 0)
    fp, ft, parts = pl.pallas_call(
        _main_kernel,
        grid=(G,),
        in_specs=[
            pl.BlockSpec((TB, DIM), blk),
            pl.BlockSpec((TB, DIM), blk),
            res((DIM, DIM)),
            res((DIM, DIM)),
            res((DIM, CAT)),
            res((CAT, DIM)),
            res((DIM, SLOT_PAD)),
            res((SLOT_PAD, DIM)),
            res((DIM, 128)),
            res((128, CAT)),
            res((CAT, 128)),
            res((128, CAT)),
            res((8, DIM)),
        ],
        out_specs=[
            pl.BlockSpec((TB, DIM), blk),
            pl.BlockSpec((TB, DIM), blk),
            pl.BlockSpec((1, 1, 128), lambda i: (i, 0, 0)),
        ],
        out_shape=[
            jax.ShapeDtypeStruct((N, DIM), f32),
            jax.ShapeDtypeStruct((N, DIM), f32),
            jax.ShapeDtypeStruct((G, 1, 128), f32),
        ],
        compiler_params=pltpu.CompilerParams(
            dimension_semantics=("parallel",),
            vmem_limit_bytes=48 * 1024 * 1024,
        ),
        name="mem_main",
    )(q2, v2, q_w.T.astype(jnp.bfloat16), v_w.T.astype(jnp.bfloat16),
      kpadt, wcat, vnpt, mvp.astype(jnp.bfloat16), sel, segt_bf, seg, segt,
      bias_pack)

    f_predict = fp.reshape(B, S, C)
    f_target_recon = ft.reshape(B, S, C)
    recon_loss = jnp.sum(parts[:, 0, 0]) / N
    return (f_predict, f_target_recon, recon_loss, closs_arr[0, 0])


# speedup vs baseline: 3.3845x; 1.1381x over previous
import functools

import numpy as np

import jax
import jax.numpy as jnp
from jax.experimental import pallas as pl
from jax.experimental.pallas import tpu as pltpu

N_SLOT = 112
N_HEAD = 8
DIM = 512
HEAD_DIM = 64
SLOT_PAD = 128
CAT = N_HEAD * SLOT_PAD
RADIUS = 16.0
EPS = 1e-5
TB = 512


def _f32dot(a, b):
    return jnp.dot(a, b, preferred_element_type=jnp.float32)


def _dot_rhs_t(a, b):
    return jax.lax.dot_general(a, b, (((1,), (1,)), ((), ())),
                               preferred_element_type=jnp.float32)


def _rownorm(x):
    ss = jnp.sum(x * x, axis=-1, keepdims=True)
    return x * jax.lax.rsqrt(jnp.maximum(ss, 1e-24))


def _ln(x, g, b):
    m = jnp.mean(x, axis=-1, keepdims=True)
    m2 = jnp.mean(x * x, axis=-1, keepdims=True)
    r = jax.lax.rsqrt(jnp.maximum(m2 - m * m, 0.0) + EPS)
    return (x - m) * r * g + b


def _prep_kernel(mkt_ref, mvp_ref, mvpt_ref, ow_ref, perm_ref,
                 kpadt_ref, wcat_ref, vnpt_ref, closs_ref,
                 kscr_ref, wscr_ref):
    mkt = mkt_ref[...]
    sst = jnp.sum(mkt * mkt, axis=0, keepdims=True)
    knt = mkt * jax.lax.rsqrt(jnp.maximum(sst, 1e-24))
    mvp = mvp_ref[...]
    for h in range(N_HEAD):
        bh = knt[:, h * N_SLOT:(h + 1) * N_SLOT]
        kscr_ref[h * HEAD_DIM:(h + 1) * HEAD_DIM, :] = jnp.pad(
            bh, ((0, 0), (h * SLOT_PAD, CAT - h * SLOT_PAD - N_SLOT)),
        ).astype(jnp.bfloat16)
        wscr_ref[h * SLOT_PAD:(h + 1) * SLOT_PAD, :] = _dot_rhs_t(
            mvp, ow_ref[:, h * DIM:(h + 1) * DIM]).astype(jnp.bfloat16)
    kpadt_ref[...] = _f32dot(kscr_ref[...], perm_ref[...]).astype(jnp.bfloat16)
    wcat_ref[...] = jax.lax.dot_general(
        perm_ref[...], wscr_ref[...], (((0,), (0,)), ((), ())),
        preferred_element_type=jnp.float32).astype(jnp.bfloat16)
    mvpt = mvpt_ref[...]
    ssv = jnp.sum(mvpt * mvpt, axis=0, keepdims=True)
    vnt = mvpt * jax.lax.rsqrt(jnp.maximum(ssv, 1e-24))
    vnpt_ref[...] = vnt.astype(jnp.bfloat16)
    gram = jax.lax.dot_general(vnt, vnt, (((0,), (0,)), ((), ())),
                               preferred_element_type=jnp.float32)
    r = jax.lax.broadcasted_iota(jnp.int32, (SLOT_PAD, SLOT_PAD), 0)
    c = jax.lax.broadcasted_iota(jnp.int32, (SLOT_PAD, SLOT_PAD), 1)
    eye = jnp.where((r == c) & (r < N_SLOT), 1.0, 0.0)
    closs_ref[...] = jnp.full((8, 128), jnp.sum(jnp.abs(eye - gram)) * 0.01,
                              jnp.float32)


def _main_kernel(q_ref, v_ref, qwt_ref, vwt_ref, kpadt_ref, wcat_ref,
                 vnpt_ref, mvp_ref, sel_ref, seg_ref,
                 bias_ref, fp_ref, ft_ref, part_ref):
    bf16 = jnp.bfloat16
    LOG2E = 1.4426950408889634
    ln1_g, ln1_b = bias_ref[3:4, :], bias_ref[4:5, :]

    def _half(r0, rows):
        sl = slice(r0, r0 + rows)
        q = q_ref[sl, :]
        v = v_ref[sl, :]

        qp = _f32dot(q.astype(bf16), qwt_ref[...]) + bias_ref[0:1, :]
        qpb = qp.astype(bf16)
        ssh = _f32dot(qpb * qpb, sel_ref[...])
        invc = jax.lax.rsqrt(jnp.maximum(ssh, 1e-24)) * (RADIUS * LOG2E)
        raw = _f32dot(qpb, kpadt_ref[...])
        e = jnp.exp2(raw * pltpu.repeat(invc, N_HEAD, axis=1))
        eb = e.astype(bf16)
        ssum = _f32dot(eb, seg_ref[...]) - 16.0
        rec = (1.0 / jnp.maximum(ssum, 1e-30)).astype(bf16)
        addr = eb * pltpu.repeat(rec, N_HEAD, axis=1)
        attn = _f32dot(addr, wcat_ref[...]) + bias_ref[2:3, :]
        fp_ref[sl, :] = _ln(q + attn, ln1_g, ln1_b)

        vp = _f32dot(v.astype(bf16), vwt_ref[...]) + bias_ref[1:2, :]
        ssv = jnp.sum(vp * vp, axis=-1, keepdims=True)
        rc = jax.lax.rsqrt(jnp.maximum(ssv, 1e-24)) * (RADIUS * LOG2E)
        raw2 = _f32dot(vp.astype(bf16), vnpt_ref[...])
        e2 = jnp.exp2(raw2 * rc)
        addr2 = e2 / (jnp.sum(e2, axis=-1, keepdims=True) - 16.0)
        ar = _f32dot(addr2.astype(bf16), mvp_ref[...])
        s_av = jnp.sum(ar * v, axis=-1, keepdims=True)
        s_aa = jnp.sum(ar * ar, axis=-1, keepdims=True)
        s_vv = jnp.sum(v * v, axis=-1, keepdims=True)
        cos = s_av * jax.lax.rsqrt(jnp.maximum(s_aa * s_vv, 1e-24))
        ft_ref[sl, :] = _ln(q + _ln(ar, bias_ref[5:6, :], bias_ref[6:7, :]),
                            ln1_g, ln1_b)
        return jnp.sum(jnp.abs(1.0 - cos))

    part = _half(0, TB)
    part_ref[...] = jnp.full((1, 1, 128), part, jnp.float32)


def kernel(query, value, mem_key, mem_value, q_w, q_b, v_w, v_b,
           out_w, out_b, ln1_g, ln1_b, ln3_g, ln3_b):
    B, S, C = query.shape
    N = B * S
    G = N // TB
    f32 = jnp.float32
    q2 = query.reshape(N, C)
    v2 = value.reshape(N, DIM)
    mvp = jnp.pad(mem_value, ((0, SLOT_PAD - N_SLOT), (0, 0)))

    co = np.arange(CAT)
    perm_np = np.zeros((CAT, CAT), np.float32)
    perm_np[co, (co % SLOT_PAD) * N_HEAD + co // SLOT_PAD] = 1.0
    perm = jnp.asarray(perm_np, jnp.bfloat16)

    kpadt, wcat, vnpt, closs_arr = pl.pallas_call(
        _prep_kernel,
        out_shape=[
            jax.ShapeDtypeStruct((DIM, CAT), jnp.bfloat16),
            jax.ShapeDtypeStruct((CAT, DIM), jnp.bfloat16),
            jax.ShapeDtypeStruct((DIM, SLOT_PAD), jnp.bfloat16),
            jax.ShapeDtypeStruct((8, 128), f32),
        ],
        scratch_shapes=[
            pltpu.VMEM((DIM, CAT), jnp.bfloat16),
            pltpu.VMEM((CAT, DIM), jnp.bfloat16),
        ],
        name="mem_prep",
    )(mem_key.T, mvp, mvp.T, out_w, perm)

    di = np.arange(DIM)
    hi = np.arange(128)
    ci = np.arange(CAT)
    sel = jnp.asarray((di[:, None] // HEAD_DIM == hi[None, :] % N_HEAD),
                      jnp.bfloat16)
    seg = jnp.asarray((ci[:, None] % N_HEAD == hi[None, :] % N_HEAD),
                      jnp.bfloat16)
    bias_pack = jnp.stack(
        [q_b, v_b, out_b, ln1_g, ln1_b, ln3_g, ln3_b, jnp.zeros_like(q_b)],
        axis=0)

    res = lambda shape: pl.BlockSpec(shape, lambda i: (0,) * len(shape))
    blk = lambda i: (i, 0)
    fp, ft, parts = pl.pallas_call(
        _main_kernel,
        grid=(G,),
        in_specs=[
            pl.BlockSpec((TB, DIM), blk),
            pl.BlockSpec((TB, DIM), blk),
            res((DIM, DIM)),
            res((DIM, DIM)),
            res((DIM, CAT)),
            res((CAT, DIM)),
            res((DIM, SLOT_PAD)),
            res((SLOT_PAD, DIM)),
            res((DIM, 128)),
            res((CAT, 128)),
            res((8, DIM)),
        ],
        out_specs=[
            pl.BlockSpec((TB, DIM), blk),
            pl.BlockSpec((TB, DIM), blk),
            pl.BlockSpec((1, 1, 128), lambda i: (i, 0, 0)),
        ],
        out_shape=[
            jax.ShapeDtypeStruct((N, DIM), f32),
            jax.ShapeDtypeStruct((N, DIM), f32),
            jax.ShapeDtypeStruct((G, 1, 128), f32),
        ],
        compiler_params=pltpu.CompilerParams(
            dimension_semantics=("parallel",),
            vmem_limit_bytes=48 * 1024 * 1024,
        ),
        name="mem_main",
    )(q2, v2, q_w.T.astype(jnp.bfloat16), v_w.T.astype(jnp.bfloat16),
      kpadt, wcat, vnpt, mvp.astype(jnp.bfloat16), sel, seg, bias_pack)

    f_predict = fp.reshape(B, S, C)
    f_target_recon = ft.reshape(B, S, C)
    recon_loss = jnp.sum(parts[:, 0, 0]) / N
    return (f_predict, f_target_recon, recon_loss, closs_arr[0, 0])
